# baseline (speedup 1.0000x reference)
import jax
import jax.numpy as jnp
from jax import lax
from jax.experimental import pallas as pl
from jax.experimental.pallas import tpu as pltpu
from jax.experimental.pallas import tpu_sc as plsc

R = 128
N = 32768
KTOP = 2048
L = 16
NV = N // L
KV = KTOP // L
CHUNK = KTOP // L
NTILES = 32

_U32 = jnp.uint32
_I32 = jnp.int32


def _u(x):
  return lax.bitcast_convert_type(x, _U32)


def _f(x):
  return lax.bitcast_convert_type(x, jnp.float32)


def _flip(u):
  sgn = u >> _U32(31)
  m = _U32(0x80000000) + sgn * _U32(0x7FFFFFFF)
  return u ^ m


def _unflip(u):
  sgn = u >> _U32(31)
  m = _U32(0x80000000) + (_U32(1) - sgn) * _U32(0x7FFFFFFF)
  return u ^ m


def _popcnt(mask):
  return plsc.all_reduce_population_count(mask)[0]


def _body(x_hbm, vout_hbm, iout_hbm, xbuf, side_i, hist, chk, cand_u,
          cand_i, cand2_u, cand2_i, dma_sem):
  info = plsc.get_sparse_core_info()
  nc = info.num_cores
  wid = lax.axis_index("s") * nc + lax.axis_index("c")
  lane = lax.iota(_I32, L)
  ones = jnp.ones((L,), _I32)
  zeros16 = jnp.zeros((L,), _I32)

  def clear_hist(nbins=256):
    @plsc.parallel_loop(0, nbins // 8)
    def cb(i):
      b = i * 8
      for j in range(8):
        hist[pl.ds((b + j) * L, L)] = zeros16

  def find_bin(k_need, nbins=256):
    nch = nbins // 16

    def outer(c, s_acc):
      base = nbins - 1 - c * 16
      for j in range(16):
        s_acc = s_acc + hist[pl.ds((base - j) * L, L)]
      chk[pl.ds(c * L, L)] = s_acc
      return s_acc
    plsc.parallel_loop(0, nch, carry=jnp.zeros((L,), _I32))(outer)

    def fc(c, carry):
      prev, cfound, cum_above = carry
      t = jnp.sum(chk[pl.ds(c * L, L)])
      hit = jnp.logical_and(cfound < 0, t >= k_need)
      cfound = jnp.where(hit, c, cfound)
      cum_above = jnp.where(hit, prev, cum_above)
      return t, cfound, cum_above
    _, cfound, cum_above = plsc.parallel_loop(
        0, nch, carry=(_I32(0), _I32(-1), _I32(0)))(fc)

    bstart = nbins - 1 - cfound * 16

    def fb(j, carry):
      cum, bfound, cabove = carry
      b = bstart - j
      s = jnp.sum(hist[pl.ds(b * L, L)])
      hit = jnp.logical_and(cum < k_need, cum + s >= k_need)
      bfound = jnp.where(hit, b, bfound)
      cabove = jnp.where(hit, cum, cabove)
      return cum + s, bfound, cabove
    _, bfound, cabove = plsc.parallel_loop(
        0, 16, carry=(cum_above, _I32(0), _I32(0)))(fb)
    return bfound, cabove

  first_row = wid * (R // NTILES)
  pltpu.sync_copy(x_hbm.at[first_row], xbuf)

  def process_row(r, _):
    row = first_row + r

    clear_hist(1024)

    @plsc.parallel_loop(0, NV, unroll=24)
    def l0(t):
      v = xbuf[pl.ds(t * L, L)]
      up = _flip(_u(v))
      xbuf[pl.ds(t * L, L)] = _f(up)
      dl = (up >> _U32(22)).astype(_I32) * L + lane
      plsc.addupdate_scatter(hist, [dl], ones)

    b3, _ = find_bin(KTOP, 1024)
    b3u = b3.astype(_U32)

    def pb(t, carry):
      off_gt, off_eq = carry
      v = xbuf[pl.ds(t * L, L)]
      up = _u(v)
      d = up >> _U32(22)
      idxv = t * L + lane
      mgt = d > b3u
      meq = d == b3u
      plsc.store_compressed(cand_u.at[pl.ds(off_gt, L)], v, mask=mgt)
      plsc.store_compressed(cand_i.at[pl.ds(off_gt, L)], idxv, mask=mgt)
      plsc.store_compressed(side_i.at[pl.ds(off_eq, L)], idxv, mask=meq)
      return off_gt + _popcnt(mgt), off_eq + _popcnt(meq)
    off_gt, m = plsc.parallel_loop(
        0, NV, unroll=24, carry=(_I32(0), _I32(0)))(pb)

    def refine(shift, carry):
      off_gt, m = carry
      clear_hist()
      nvec = (m + L - 1) // L

      def hl(t):
        valid = (t * L + lane) < m
        iv = side_i[pl.ds(t * L, L)]
        v = plsc.load_gather(xbuf, [iv], mask=valid)
        up = _u(v)
        dl = ((up >> _U32(shift)) & _U32(0xFF)).astype(_I32) * L + lane
        plsc.addupdate_scatter(hist, [dl], ones, mask=valid)
      plsc.parallel_loop(0, nvec, unroll=4)(hl)

      b, _ = find_bin(KTOP - off_gt)
      bu = b.astype(_U32)

      def cl(t, carry):
        o_gt, o_eq = carry
        valid = (t * L + lane) < m
        iv = side_i[pl.ds(t * L, L)]
        v = plsc.load_gather(xbuf, [iv], mask=valid)
        up = _u(v)
        d = (up >> _U32(shift)) & _U32(0xFF)
        mgt = jnp.logical_and(d > bu, valid)
        meq = jnp.logical_and(d == bu, valid)
        plsc.store_compressed(cand_u.at[pl.ds(o_gt, L)], v, mask=mgt)
        plsc.store_compressed(cand_i.at[pl.ds(o_gt, L)], iv, mask=mgt)
        plsc.store_compressed(side_i.at[pl.ds(o_eq, L)], iv, mask=meq)
        return o_gt + _popcnt(mgt), o_eq + _popcnt(meq)
      return lax.fori_loop(0, nvec, cl, (off_gt, 0))

    off_gt, m = refine(14, (off_gt, m))
    off_gt, m = refine(6, (off_gt, m))

    clear_hist(64)
    nvec = (m + L - 1) // L

    def h3(t):
      valid = (t * L + lane) < m
      iv = side_i[pl.ds(t * L, L)]
      v = plsc.load_gather(xbuf, [iv], mask=valid)
      up = _u(v)
      dl = (up & _U32(0x3F)).astype(_I32) * L + lane
      plsc.addupdate_scatter(hist, [dl], ones, mask=valid)
    plsc.parallel_loop(0, nvec, unroll=4)(h3)

    b0, cab3 = find_bin(KTOP - off_gt, 64)
    b0u = b0.astype(_U32)
    need_eq = KTOP - off_gt - cab3
    eq_base = off_gt + cab3

    def c3(t, carry):
      o_gt, taken = carry
      valid = (t * L + lane) < m
      iv = side_i[pl.ds(t * L, L)]
      v = plsc.load_gather(xbuf, [iv], mask=valid)
      up = _u(v)
      d = up & _U32(0x3F)
      mgt = jnp.logical_and(d > b0u, valid)
      meq = jnp.logical_and(d == b0u, valid)
      cs = plsc.cumsum(meq.astype(_I32))
      keep = jnp.logical_and(meq, (taken + cs) <= need_eq)
      plsc.store_compressed(cand_u.at[pl.ds(o_gt, L)], v, mask=mgt)
      plsc.store_compressed(cand_i.at[pl.ds(o_gt, L)], iv, mask=mgt)
      plsc.store_compressed(cand_u.at[pl.ds(eq_base + taken, L)], v, mask=keep)
      plsc.store_compressed(cand_i.at[pl.ds(eq_base + taken, L)], iv,
                            mask=keep)
      return o_gt + _popcnt(mgt), taken + _popcnt(keep)
    plsc.parallel_loop(0, nvec, carry=(off_gt, _I32(0)))(c3)

    nxt = jnp.minimum(row + 1, R - 1)
    cp = pltpu.make_async_copy(x_hbm.at[nxt], xbuf, dma_sem)
    cp.start()

    chunk_base = lane * (CHUNK + 1)

    @plsc.parallel_loop(0, KV, unroll=2)
    def rl(t):
      base_p = t * L + t // 8
      cand2_u[pl.ds(base_p, L)] = cand_u[pl.ds(t * L, L)]
      cand2_i[pl.ds(base_p, L)] = cand_i[pl.ds(t * L, L)]

    def radix_pass(shift, src_u, src_i, dst_u, dst_i, dst_padded):
      clear_hist()

      @plsc.parallel_loop(0, CHUNK, unroll=8)
      def hp(t):
        g = chunk_base + t
        kv = plsc.load_gather(src_u, [g])
        dl = ((_u(kv) >> _U32(shift)) & _U32(0xFF)).astype(_I32) * L + lane
        plsc.addupdate_scatter(hist, [dl], ones)

      def op(i, total):
        b = 255 - i
        incl = plsc.cumsum(hist[pl.ds(b * L, L)])
        hist[pl.ds(b * L, L)] = incl + total
        return total + incl[L - 1]
      plsc.parallel_loop(0, 256, unroll=32, carry=_I32(0))(op)

      def sp(i, _):
        t = CHUNK - 1 - i
        g = chunk_base + t
        kv = plsc.load_gather(src_u, [g])
        iv = plsc.load_gather(src_i, [g])
        dl = ((_u(kv) >> _U32(shift)) & _U32(0xFF)).astype(_I32) * L + lane
        pos = plsc.load_gather(hist, [dl]) - 1
        plsc.addupdate_scatter(hist, [dl], -ones)
        if dst_padded:
          pos = pos + (pos >> 7)
        plsc.store_scatter(dst_u, [pos], kv)
        plsc.store_scatter(dst_i, [pos], iv)
        return 0
      lax.fori_loop(0, CHUNK, sp, 0, unroll=16)

    radix_pass(0, cand2_u, cand2_i, cand_u, cand_i, True)
    radix_pass(8, cand_u, cand_i, cand2_u, cand2_i, True)
    radix_pass(16, cand2_u, cand2_i, cand_u, cand_i, True)
    radix_pass(24, cand_u, cand_i, cand2_u, cand2_i, False)

    def uf(t, _):
      up = _u(cand2_u[pl.ds(t * L, L)])
      cand_u[pl.ds(t * L, L)] = _f(_unflip(up))
      return 0
    lax.fori_loop(0, KV, uf, 0, unroll=2)

    pltpu.sync_copy(cand_u.at[pl.ds(0, KTOP)], vout_hbm.at[row])
    pltpu.sync_copy(cand2_i.at[pl.ds(0, KTOP)], iout_hbm.at[row])
    cp.wait()
    return 0

  lax.fori_loop(0, R // NTILES, process_row, 0)


@jax.jit
def _topk(x):
  mesh = plsc.VectorSubcoreMesh(core_axis_name="c", subcore_axis_name="s")
  fn = pl.kernel(
      _body,
      out_type=(
          jax.ShapeDtypeStruct((R, KTOP), jnp.float32),
          jax.ShapeDtypeStruct((R, KTOP), jnp.int32),
      ),
      mesh=mesh,
      compiler_params=pltpu.CompilerParams(needs_layout_passes=False),
      scratch_types=[
          pltpu.VMEM((N,), jnp.float32),
          pltpu.VMEM((N,), jnp.int32),
          pltpu.VMEM((1024 * L,), jnp.int32),
          pltpu.VMEM((64 * L,), jnp.int32),
          pltpu.VMEM((KTOP + L,), jnp.float32),
          pltpu.VMEM((KTOP + L,), jnp.int32),
          pltpu.VMEM((KTOP + L,), jnp.float32),
          pltpu.VMEM((KTOP + L,), jnp.int32),
          pltpu.SemaphoreType.DMA,
      ],
  )
  return fn(x)


def kernel(X, K):
  values, indices = _topk(X)
  return values, indices + (jnp.asarray(K, indices.dtype) - KTOP)


# speedup vs baseline: 1.3388x; 1.3388x over previous
import jax
import jax.numpy as jnp
from jax import lax
from jax.experimental import pallas as pl
from jax.experimental.pallas import tpu as pltpu
from jax.experimental.pallas import tpu_sc as plsc

R = 128
N = 32768
KTOP = 2048
L = 16
NV = N // L
KV = KTOP // L
CHUNK = KTOP // L
NTILES = 32

_U32 = jnp.uint32
_I32 = jnp.int32


def _u(x):
  return lax.bitcast_convert_type(x, _U32)


def _f(x):
  return lax.bitcast_convert_type(x, jnp.float32)


def _flip(u):
  sgn = u >> _U32(31)
  m = _U32(0x80000000) + sgn * _U32(0x7FFFFFFF)
  return u ^ m


def _unflip(u):
  sgn = u >> _U32(31)
  m = _U32(0x80000000) + (_U32(1) - sgn) * _U32(0x7FFFFFFF)
  return u ^ m


def _popcnt(mask):
  return plsc.all_reduce_population_count(mask)[0]


def _body(x_hbm, vout_hbm, iout_hbm, xbuf, side_i, hist, chk, cand_u,
          cand_i, cand2_u, cand2_i, dma_sem):
  info = plsc.get_sparse_core_info()
  nc = info.num_cores
  wid = lax.axis_index("s") * nc + lax.axis_index("c")
  lane = lax.iota(_I32, L)
  ones = jnp.ones((L,), _I32)
  zeros16 = jnp.zeros((L,), _I32)

  def clear_hist(nbins=256):
    @plsc.parallel_loop(0, nbins // 8)
    def cb(i):
      b = i * 8
      for j in range(8):
        hist[pl.ds((b + j) * L, L)] = zeros16

  def find_bin(k_need, nbins=256):
    nch = nbins // 16

    def outer(c, s_acc):
      base = nbins - 1 - c * 16
      for j in range(16):
        s_acc = s_acc + hist[pl.ds((base - j) * L, L)]
      chk[pl.ds(c * L, L)] = s_acc
      return s_acc
    plsc.parallel_loop(0, nch, carry=jnp.zeros((L,), _I32))(outer)

    def fc(c, carry):
      prev, cfound, cum_above = carry
      t = jnp.sum(chk[pl.ds(c * L, L)])
      hit = jnp.logical_and(cfound < 0, t >= k_need)
      cfound = jnp.where(hit, c, cfound)
      cum_above = jnp.where(hit, prev, cum_above)
      return t, cfound, cum_above
    _, cfound, cum_above = plsc.parallel_loop(
        0, nch, carry=(_I32(0), _I32(-1), _I32(0)))(fc)

    bstart = nbins - 1 - cfound * 16

    def fb(j, carry):
      cum, bfound, cabove = carry
      b = bstart - j
      s = jnp.sum(hist[pl.ds(b * L, L)])
      hit = jnp.logical_and(cum < k_need, cum + s >= k_need)
      bfound = jnp.where(hit, b, bfound)
      cabove = jnp.where(hit, cum, cabove)
      return cum + s, bfound, cabove
    _, bfound, cabove = plsc.parallel_loop(
        0, 16, carry=(cum_above, _I32(0), _I32(0)))(fb)
    return bfound, cabove

  first_row = wid * (R // NTILES)
  pltpu.sync_copy(x_hbm.at[first_row], xbuf)

  def process_row(r, _):
    row = first_row + r

    clear_hist(1024)

    @plsc.parallel_loop(0, NV, unroll=16)
    def l0(t):
      v = xbuf[pl.ds(t * L, L)]
      up = _flip(_u(v))
      xbuf[pl.ds(t * L, L)] = _f(up)
      dl = (up >> _U32(22)).astype(_I32) * L + lane
      plsc.addupdate_scatter(hist, [dl], ones)

    b3, _ = find_bin(KTOP, 1024)
    b3u = b3.astype(_U32)

    def pb(t, carry):
      off_gt, off_eq = carry
      v = xbuf[pl.ds(t * L, L)]
      up = _u(v)
      d = up >> _U32(22)
      idxv = t * L + lane
      mgt = d > b3u
      meq = d == b3u
      plsc.store_compressed(cand_u.at[pl.ds(off_gt, L)], v, mask=mgt)
      plsc.store_compressed(cand_i.at[pl.ds(off_gt, L)], idxv, mask=mgt)
      plsc.store_compressed(side_i.at[pl.ds(off_eq, L)], idxv, mask=meq)
      return off_gt + _popcnt(mgt), off_eq + _popcnt(meq)
    off_gt, m = plsc.parallel_loop(
        0, NV, unroll=16, carry=(_I32(0), _I32(0)))(pb)

    def refine(shift, carry):
      off_gt, m = carry
      clear_hist()
      nvec = (m + L - 1) // L

      def hl(t):
        valid = (t * L + lane) < m
        iv = side_i[pl.ds(t * L, L)]
        v = plsc.load_gather(xbuf, [iv], mask=valid)
        up = _u(v)
        dl = ((up >> _U32(shift)) & _U32(0xFF)).astype(_I32) * L + lane
        plsc.addupdate_scatter(hist, [dl], ones, mask=valid)
      plsc.parallel_loop(0, nvec, unroll=4)(hl)

      b, _ = find_bin(KTOP - off_gt)
      bu = b.astype(_U32)

      def cl(t, carry):
        o_gt, o_eq = carry
        valid = (t * L + lane) < m
        iv = side_i[pl.ds(t * L, L)]
        v = plsc.load_gather(xbuf, [iv], mask=valid)
        up = _u(v)
        d = (up >> _U32(shift)) & _U32(0xFF)
        mgt = jnp.logical_and(d > bu, valid)
        meq = jnp.logical_and(d == bu, valid)
        plsc.store_compressed(cand_u.at[pl.ds(o_gt, L)], v, mask=mgt)
        plsc.store_compressed(cand_i.at[pl.ds(o_gt, L)], iv, mask=mgt)
        plsc.store_compressed(side_i.at[pl.ds(o_eq, L)], iv, mask=meq)
        return o_gt + _popcnt(mgt), o_eq + _popcnt(meq)
      return lax.fori_loop(0, nvec, cl, (off_gt, 0))

    off_gt, m = refine(14, (off_gt, m))
    off_gt, m = refine(6, (off_gt, m))

    clear_hist(64)
    nvec = (m + L - 1) // L

    def h3(t):
      valid = (t * L + lane) < m
      iv = side_i[pl.ds(t * L, L)]
      v = plsc.load_gather(xbuf, [iv], mask=valid)
      up = _u(v)
      dl = (up & _U32(0x3F)).astype(_I32) * L + lane
      plsc.addupdate_scatter(hist, [dl], ones, mask=valid)
    plsc.parallel_loop(0, nvec, unroll=4)(h3)

    b0, cab3 = find_bin(KTOP - off_gt, 64)
    b0u = b0.astype(_U32)
    need_eq = KTOP - off_gt - cab3
    eq_base = off_gt + cab3

    def c3(t, carry):
      o_gt, taken = carry
      valid = (t * L + lane) < m
      iv = side_i[pl.ds(t * L, L)]
      v = plsc.load_gather(xbuf, [iv], mask=valid)
      up = _u(v)
      d = up & _U32(0x3F)
      mgt = jnp.logical_and(d > b0u, valid)
      meq = jnp.logical_and(d == b0u, valid)
      cs = plsc.cumsum(meq.astype(_I32))
      keep = jnp.logical_and(meq, (taken + cs) <= need_eq)
      plsc.store_compressed(cand_u.at[pl.ds(o_gt, L)], v, mask=mgt)
      plsc.store_compressed(cand_i.at[pl.ds(o_gt, L)], iv, mask=mgt)
      plsc.store_compressed(cand_u.at[pl.ds(eq_base + taken, L)], v, mask=keep)
      plsc.store_compressed(cand_i.at[pl.ds(eq_base + taken, L)], iv,
                            mask=keep)
      return o_gt + _popcnt(mgt), taken + _popcnt(keep)
    plsc.parallel_loop(0, nvec, carry=(off_gt, _I32(0)))(c3)

    nxt = jnp.minimum(row + 1, R - 1)
    cp = pltpu.make_async_copy(x_hbm.at[nxt], xbuf, dma_sem)
    cp.start()

    chunk_base = lane * (CHUNK + 1)

    @plsc.parallel_loop(0, KV, unroll=2)
    def rl(t):
      base_p = t * L + t // 8
      cand2_u[pl.ds(base_p, L)] = cand_u[pl.ds(t * L, L)]
      cand2_i[pl.ds(base_p, L)] = cand_i[pl.ds(t * L, L)]

    def radix_pass(shift, src_u, src_i, dst_u, dst_i, dst_padded):
      clear_hist()

      @plsc.parallel_loop(0, CHUNK, unroll=8)
      def hp(t):
        g = chunk_base + t
        kv = plsc.load_gather(src_u, [g])
        dl = ((_u(kv) >> _U32(shift)) & _U32(0xFF)).astype(_I32) * L + lane
        plsc.addupdate_scatter(hist, [dl], ones)

      def op(i, total):
        b = 255 - i
        incl = plsc.cumsum(hist[pl.ds(b * L, L)])
        hist[pl.ds(b * L, L)] = incl + total
        return total + incl[L - 1]
      plsc.parallel_loop(0, 256, unroll=32, carry=_I32(0))(op)

      def sp(i, _):
        t = CHUNK - 1 - i
        g = chunk_base + t
        kv = plsc.load_gather(src_u, [g])
        iv = plsc.load_gather(src_i, [g])
        dl = ((_u(kv) >> _U32(shift)) & _U32(0xFF)).astype(_I32) * L + lane
        pos = plsc.load_gather(hist, [dl]) - 1
        plsc.addupdate_scatter(hist, [dl], -ones)
        if dst_padded:
          pos = pos + (pos >> 7)
        plsc.store_scatter(dst_u, [pos], kv)
        plsc.store_scatter(dst_i, [pos], iv)
        return 0
      lax.fori_loop(0, CHUNK, sp, 0, unroll=16)

    radix_pass(0, cand2_u, cand2_i, cand_u, cand_i, True)
    radix_pass(8, cand_u, cand_i, cand2_u, cand2_i, True)
    radix_pass(16, cand2_u, cand2_i, cand_u, cand_i, True)
    radix_pass(24, cand_u, cand_i, cand2_u, cand2_i, False)

    def uf(t, _):
      up = _u(cand2_u[pl.ds(t * L, L)])
      cand_u[pl.ds(t * L, L)] = _f(_unflip(up))
      return 0
    lax.fori_loop(0, KV, uf, 0, unroll=2)

    pltpu.sync_copy(cand_u.at[pl.ds(0, KTOP)], vout_hbm.at[row])
    pltpu.sync_copy(cand2_i.at[pl.ds(0, KTOP)], iout_hbm.at[row])
    cp.wait()
    return 0

  lax.fori_loop(0, R // NTILES, process_row, 0)


@jax.jit
def _topk(x):
  mesh = plsc.VectorSubcoreMesh(core_axis_name="c", subcore_axis_name="s")
  fn = pl.kernel(
      _body,
      out_type=(
          jax.ShapeDtypeStruct((R, KTOP), jnp.float32),
          jax.ShapeDtypeStruct((R, KTOP), jnp.int32),
      ),
      mesh=mesh,
      compiler_params=pltpu.CompilerParams(needs_layout_passes=False),
      scratch_types=[
          pltpu.VMEM((N,), jnp.float32),
          pltpu.VMEM((N,), jnp.int32),
          pltpu.VMEM((1024 * L,), jnp.int32),
          pltpu.VMEM((64 * L,), jnp.int32),
          pltpu.VMEM((KTOP + L,), jnp.float32),
          pltpu.VMEM((KTOP + L,), jnp.int32),
          pltpu.VMEM((KTOP + L,), jnp.float32),
          pltpu.VMEM((KTOP + L,), jnp.int32),
          pltpu.SemaphoreType.DMA,
      ],
  )
  return fn(x)


def kernel(X, K):
  values, indices = _topk(X)
  return values, indices + (jnp.asarray(K, indices.dtype) - KTOP)


# speedup vs baseline: 1.3416x; 1.0022x over previous
import jax
import jax.numpy as jnp
from jax import lax
from jax.experimental import pallas as pl
from jax.experimental.pallas import tpu as pltpu
from jax.experimental.pallas import tpu_sc as plsc

R = 128
N = 32768
KTOP = 2048
L = 16
NV = N // L
KV = KTOP // L
CHUNK = KTOP // L
NTILES = 32

_U32 = jnp.uint32
_I32 = jnp.int32


def _u(x):
  return lax.bitcast_convert_type(x, _U32)


def _f(x):
  return lax.bitcast_convert_type(x, jnp.float32)


def _flip(u):
  sgn = u >> _U32(31)
  m = _U32(0x80000000) + sgn * _U32(0x7FFFFFFF)
  return u ^ m


def _unflip(u):
  sgn = u >> _U32(31)
  m = _U32(0x80000000) + (_U32(1) - sgn) * _U32(0x7FFFFFFF)
  return u ^ m


def _popcnt(mask):
  return plsc.all_reduce_population_count(mask)[0]


def _body(x_hbm, vout_hbm, iout_hbm, xbuf, side_i, hist, chk, cand_u,
          cand_i, cand2_u, cand2_i, dma_sem):
  info = plsc.get_sparse_core_info()
  nc = info.num_cores
  wid = lax.axis_index("s") * nc + lax.axis_index("c")
  lane = lax.iota(_I32, L)
  ones = jnp.ones((L,), _I32)
  zeros16 = jnp.zeros((L,), _I32)

  def clear_hist(nbins=256):
    @plsc.parallel_loop(0, nbins // 8)
    def cb(i):
      b = i * 8
      for j in range(8):
        hist[pl.ds((b + j) * L, L)] = zeros16

  def find_bin(k_need, nbins=256):
    nch = nbins // 16

    def outer(c, s_acc):
      base = nbins - 1 - c * 16
      for j in range(16):
        s_acc = s_acc + hist[pl.ds((base - j) * L, L)]
      chk[pl.ds(c * L, L)] = s_acc
      return s_acc
    plsc.parallel_loop(0, nch, carry=jnp.zeros((L,), _I32))(outer)

    def fc(c, carry):
      prev, cfound, cum_above = carry
      t = jnp.sum(chk[pl.ds(c * L, L)])
      hit = jnp.logical_and(cfound < 0, t >= k_need)
      cfound = jnp.where(hit, c, cfound)
      cum_above = jnp.where(hit, prev, cum_above)
      return t, cfound, cum_above
    _, cfound, cum_above = plsc.parallel_loop(
        0, nch, carry=(_I32(0), _I32(-1), _I32(0)))(fc)

    bstart = nbins - 1 - cfound * 16

    def fb(j, carry):
      cum, bfound, cabove = carry
      b = bstart - j
      s = jnp.sum(hist[pl.ds(b * L, L)])
      hit = jnp.logical_and(cum < k_need, cum + s >= k_need)
      bfound = jnp.where(hit, b, bfound)
      cabove = jnp.where(hit, cum, cabove)
      return cum + s, bfound, cabove
    _, bfound, cabove = plsc.parallel_loop(
        0, 16, carry=(cum_above, _I32(0), _I32(0)))(fb)
    return bfound, cabove

  first_row = wid * (R // NTILES)
  pltpu.sync_copy(x_hbm.at[first_row], xbuf)

  def process_row(r, _):
    row = first_row + r

    clear_hist(1024)

    @plsc.parallel_loop(0, NV, unroll=16)
    def l0(t):
      v = xbuf[pl.ds(t * L, L)]
      up = _flip(_u(v))
      xbuf[pl.ds(t * L, L)] = _f(up)
      dl = (up >> _U32(22)).astype(_I32) * L + lane
      plsc.addupdate_scatter(hist, [dl], ones)

    b3, _ = find_bin(KTOP, 1024)
    b3u = b3.astype(_U32)

    def pb(t, carry):
      off_gt, off_eq = carry
      v = xbuf[pl.ds(t * L, L)]
      up = _u(v)
      d = up >> _U32(22)
      idxv = t * L + lane
      mgt = d > b3u
      meq = d == b3u
      plsc.store_compressed(cand_u.at[pl.ds(off_gt, L)], v, mask=mgt)
      plsc.store_compressed(cand_i.at[pl.ds(off_gt, L)], idxv, mask=mgt)
      plsc.store_compressed(side_i.at[pl.ds(off_eq, L)], idxv, mask=meq)
      return off_gt + _popcnt(mgt), off_eq + _popcnt(meq)
    off_gt, m = plsc.parallel_loop(
        0, NV, unroll=16, carry=(_I32(0), _I32(0)))(pb)

    def refine(shift, carry):
      off_gt, m = carry
      clear_hist()
      nvec = (m + L - 1) // L

      def hl(t):
        valid = (t * L + lane) < m
        iv = side_i[pl.ds(t * L, L)]
        v = plsc.load_gather(xbuf, [iv], mask=valid)
        up = _u(v)
        dl = ((up >> _U32(shift)) & _U32(0xFF)).astype(_I32) * L + lane
        plsc.addupdate_scatter(hist, [dl], ones, mask=valid)
      plsc.parallel_loop(0, nvec, unroll=4)(hl)

      b, _ = find_bin(KTOP - off_gt)
      bu = b.astype(_U32)

      def cl(t, carry):
        o_gt, o_eq = carry
        valid = (t * L + lane) < m
        iv = side_i[pl.ds(t * L, L)]
        v = plsc.load_gather(xbuf, [iv], mask=valid)
        up = _u(v)
        d = (up >> _U32(shift)) & _U32(0xFF)
        mgt = jnp.logical_and(d > bu, valid)
        meq = jnp.logical_and(d == bu, valid)
        plsc.store_compressed(cand_u.at[pl.ds(o_gt, L)], v, mask=mgt)
        plsc.store_compressed(cand_i.at[pl.ds(o_gt, L)], iv, mask=mgt)
        plsc.store_compressed(side_i.at[pl.ds(o_eq, L)], iv, mask=meq)
        return o_gt + _popcnt(mgt), o_eq + _popcnt(meq)
      return lax.fori_loop(0, nvec, cl, (off_gt, 0))

    off_gt, m = refine(14, (off_gt, m))
    off_gt, m = refine(6, (off_gt, m))

    clear_hist(64)
    nvec = (m + L - 1) // L

    def h3(t):
      valid = (t * L + lane) < m
      iv = side_i[pl.ds(t * L, L)]
      v = plsc.load_gather(xbuf, [iv], mask=valid)
      up = _u(v)
      dl = (up & _U32(0x3F)).astype(_I32) * L + lane
      plsc.addupdate_scatter(hist, [dl], ones, mask=valid)
    plsc.parallel_loop(0, nvec, unroll=4)(h3)

    b0, cab3 = find_bin(KTOP - off_gt, 64)
    b0u = b0.astype(_U32)
    need_eq = KTOP - off_gt - cab3
    eq_base = off_gt + cab3

    def c3(t, carry):
      o_gt, taken = carry
      valid = (t * L + lane) < m
      iv = side_i[pl.ds(t * L, L)]
      v = plsc.load_gather(xbuf, [iv], mask=valid)
      up = _u(v)
      d = up & _U32(0x3F)
      mgt = jnp.logical_and(d > b0u, valid)
      meq = jnp.logical_and(d == b0u, valid)
      cs = plsc.cumsum(meq.astype(_I32))
      keep = jnp.logical_and(meq, (taken + cs) <= need_eq)
      plsc.store_compressed(cand_u.at[pl.ds(o_gt, L)], v, mask=mgt)
      plsc.store_compressed(cand_i.at[pl.ds(o_gt, L)], iv, mask=mgt)
      plsc.store_compressed(cand_u.at[pl.ds(eq_base + taken, L)], v, mask=keep)
      plsc.store_compressed(cand_i.at[pl.ds(eq_base + taken, L)], iv,
                            mask=keep)
      return o_gt + _popcnt(mgt), taken + _popcnt(keep)
    plsc.parallel_loop(0, nvec, carry=(off_gt, _I32(0)))(c3)

    nxt = jnp.minimum(row + 1, R - 1)
    cp = pltpu.make_async_copy(x_hbm.at[nxt], xbuf, dma_sem)
    cp.start()

    chunk_base = lane * (CHUNK + 1)

    @plsc.parallel_loop(0, KV, unroll=2)
    def rl(t):
      base_p = t * L + t // 8
      cand2_u[pl.ds(base_p, L)] = cand_u[pl.ds(t * L, L)]
      cand2_i[pl.ds(base_p, L)] = cand_i[pl.ds(t * L, L)]

    def radix_pass(shift, src_u, src_i, dst_u, dst_i, dst_padded):
      clear_hist()

      @plsc.parallel_loop(0, CHUNK, unroll=4)
      def hp(t):
        g = chunk_base + t
        kv = plsc.load_gather(src_u, [g])
        dl = ((_u(kv) >> _U32(shift)) & _U32(0xFF)).astype(_I32) * L + lane
        plsc.addupdate_scatter(hist, [dl], ones)

      def op(i, total):
        b = 255 - i
        incl = plsc.cumsum(hist[pl.ds(b * L, L)])
        hist[pl.ds(b * L, L)] = incl + total
        return total + incl[L - 1]
      plsc.parallel_loop(0, 256, unroll=16, carry=_I32(0))(op)

      def sp(i, _):
        t = CHUNK - 1 - i
        g = chunk_base + t
        kv = plsc.load_gather(src_u, [g])
        iv = plsc.load_gather(src_i, [g])
        dl = ((_u(kv) >> _U32(shift)) & _U32(0xFF)).astype(_I32) * L + lane
        pos = plsc.load_gather(hist, [dl]) - 1
        plsc.addupdate_scatter(hist, [dl], -ones)
        if dst_padded:
          pos = pos + (pos >> 7)
        plsc.store_scatter(dst_u, [pos], kv)
        plsc.store_scatter(dst_i, [pos], iv)
        return 0
      lax.fori_loop(0, CHUNK, sp, 0, unroll=16)

    radix_pass(0, cand2_u, cand2_i, cand_u, cand_i, True)
    radix_pass(8, cand_u, cand_i, cand2_u, cand2_i, True)
    radix_pass(16, cand2_u, cand2_i, cand_u, cand_i, True)
    radix_pass(24, cand_u, cand_i, cand2_u, cand2_i, False)

    def uf(t, _):
      up = _u(cand2_u[pl.ds(t * L, L)])
      cand_u[pl.ds(t * L, L)] = _f(_unflip(up))
      return 0
    lax.fori_loop(0, KV, uf, 0, unroll=2)

    pltpu.sync_copy(cand_u.at[pl.ds(0, KTOP)], vout_hbm.at[row])
    pltpu.sync_copy(cand2_i.at[pl.ds(0, KTOP)], iout_hbm.at[row])
    cp.wait()
    return 0

  lax.fori_loop(0, R // NTILES, process_row, 0)


@jax.jit
def _topk(x):
  mesh = plsc.VectorSubcoreMesh(core_axis_name="c", subcore_axis_name="s")
  fn = pl.kernel(
      _body,
      out_type=(
          jax.ShapeDtypeStruct((R, KTOP), jnp.float32),
          jax.ShapeDtypeStruct((R, KTOP), jnp.int32),
      ),
      mesh=mesh,
      compiler_params=pltpu.CompilerParams(needs_layout_passes=False),
      scratch_types=[
          pltpu.VMEM((N,), jnp.float32),
          pltpu.VMEM((N,), jnp.int32),
          pltpu.VMEM((1024 * L,), jnp.int32),
          pltpu.VMEM((64 * L,), jnp.int32),
          pltpu.VMEM((KTOP + L,), jnp.float32),
          pltpu.VMEM((KTOP + L,), jnp.int32),
          pltpu.VMEM((KTOP + L,), jnp.float32),
          pltpu.VMEM((KTOP + L,), jnp.int32),
          pltpu.SemaphoreType.DMA,
      ],
  )
  return fn(x)


def kernel(X, K):
  values, indices = _topk(X)
  return values, indices + (jnp.asarray(K, indices.dtype) - KTOP)
